# baseline (speedup 1.0000x reference)
import jax
import jax.numpy as jnp
from jax.experimental import pallas as pl

BEAT_LEN = 128
TOKEN_DIM = 64


def _proj_kernel(x_ref, wt_ref, b_ref, o_ref, bi_ref):
    BL = x_ref.shape[0] * x_ref.shape[1]
    N = x_ref.shape[2] // BEAT_LEN
    x = x_ref[...].reshape(BL * N, BEAT_LEN)
    y = (
        jnp.dot(x, wt_ref[...], preferred_element_type=jnp.float32)
        + b_ref[...]
    )
    o_ref[...] = y.reshape(o_ref.shape)

    @pl.when(pl.program_id(0) == 0)
    def _():
        bi_ref[...] = jnp.full(bi_ref.shape, float(BEAT_LEN), dtype=jnp.float32)


@jax.jit
def _run(ecg, W, b):
    B, L, T = ecg.shape
    N = T // BEAT_LEN
    wt = W.T
    b2 = b.reshape(1, TOKEN_DIM)

    X, bi = pl.pallas_call(
        _proj_kernel,
        grid=(B,),
        in_specs=[
            pl.BlockSpec((1, L, T), lambda i: (i, 0, 0)),
            pl.BlockSpec((BEAT_LEN, TOKEN_DIM), lambda i: (0, 0)),
            pl.BlockSpec((1, TOKEN_DIM), lambda i: (0, 0)),
        ],
        out_specs=[
            pl.BlockSpec((1, L, N, TOKEN_DIM), lambda i: (i, 0, 0, 0)),
            pl.BlockSpec((B, N), lambda i: (0, 0)),
        ],
        out_shape=[
            jax.ShapeDtypeStruct((B, L, N, TOKEN_DIM), jnp.float32),
            jax.ShapeDtypeStruct((B, N), jnp.float32),
        ],
    )(ecg, wt, b2)

    return (X, bi)


def kernel(ecg, W, b):
    return _run(ecg, W, b)


# speedup vs baseline: 1.3269x; 1.3269x over previous
import jax
import jax.numpy as jnp
from jax.experimental import pallas as pl
from jax.experimental.pallas import tpu as pltpu

BEAT_LEN = 128
TOKEN_DIM = 64


def _proj_kernel(x_hbm, wt_ref, b_ref, o_hbm, bi_ref, x_vmem, y_vmem,
                 in_sems, out_sems):
    B, L, T = x_hbm.shape
    N = T // BEAT_LEN
    for i in range(B):
        pltpu.make_async_copy(x_hbm.at[i], x_vmem.at[i], in_sems.at[i]).start()
    wt = wt_ref[...]
    bias = b_ref[...]
    for i in range(B):
        pltpu.make_async_copy(x_hbm.at[i], x_vmem.at[i], in_sems.at[i]).wait()
        x = x_vmem[i].reshape(L * N, BEAT_LEN)
        y = jnp.dot(x, wt, preferred_element_type=jnp.float32) + bias
        y_vmem[i] = y.reshape(L, N, TOKEN_DIM)
        pltpu.make_async_copy(y_vmem.at[i], o_hbm.at[i], out_sems.at[i]).start()
    bi_ref[...] = jnp.full(bi_ref.shape, float(BEAT_LEN), dtype=jnp.float32)
    for i in range(B):
        pltpu.make_async_copy(y_vmem.at[i], o_hbm.at[i], out_sems.at[i]).wait()


@jax.jit
def _run(ecg, W, b):
    B, L, T = ecg.shape
    N = T // BEAT_LEN
    wt = W.T
    b2 = b.reshape(1, TOKEN_DIM)

    X, bi = pl.pallas_call(
        _proj_kernel,
        in_specs=[
            pl.BlockSpec(memory_space=pl.ANY),
            pl.BlockSpec((BEAT_LEN, TOKEN_DIM), lambda: (0, 0)),
            pl.BlockSpec((1, TOKEN_DIM), lambda: (0, 0)),
        ],
        out_specs=[
            pl.BlockSpec(memory_space=pl.ANY),
            pl.BlockSpec((B, N), lambda: (0, 0)),
        ],
        out_shape=[
            jax.ShapeDtypeStruct((B, L, N, TOKEN_DIM), jnp.float32),
            jax.ShapeDtypeStruct((B, N), jnp.float32),
        ],
        scratch_shapes=[
            pltpu.VMEM((B, L, T), jnp.float32),
            pltpu.VMEM((B, L, N, TOKEN_DIM), jnp.float32),
            pltpu.SemaphoreType.DMA((B,)),
            pltpu.SemaphoreType.DMA((B,)),
        ],
    )(ecg, wt, b2)

    return (X, bi)


def kernel(ecg, W, b):
    return _run(ecg, W, b)


# speedup vs baseline: 1.4817x; 1.1167x over previous
import jax
import jax.numpy as jnp
from jax.experimental import pallas as pl
from jax.experimental.pallas import tpu as pltpu

BEAT_LEN = 128
TOKEN_DIM = 64


def _proj_kernel(x_ref, wt_ref, b_ref, o_ref, bi_ref):
    o_ref[...] = (
        jnp.dot(x_ref[...], wt_ref[...], preferred_element_type=jnp.float32)
        + b_ref[...]
    )

    @pl.when(pl.program_id(0) == 0)
    def _():
        bi_ref[...] = jnp.full(bi_ref.shape, float(BEAT_LEN), dtype=jnp.float32)


@jax.jit
def _run(ecg, W, b):
    B, L, T = ecg.shape
    N = T // BEAT_LEN
    M = B * L * N
    x = ecg.reshape(M, BEAT_LEN)
    wt = W.T
    b2 = b.reshape(1, TOKEN_DIM)

    BLK_M = 1536
    out, bi = pl.pallas_call(
        _proj_kernel,
        grid=(M // BLK_M,),
        in_specs=[
            pl.BlockSpec((BLK_M, BEAT_LEN), lambda i: (i, 0)),
            pl.BlockSpec((BEAT_LEN, TOKEN_DIM), lambda i: (0, 0)),
            pl.BlockSpec((1, TOKEN_DIM), lambda i: (0, 0)),
        ],
        out_specs=[
            pl.BlockSpec((BLK_M, TOKEN_DIM), lambda i: (i, 0)),
            pl.BlockSpec((B, N), lambda i: (0, 0)),
        ],
        out_shape=[
            jax.ShapeDtypeStruct((M, TOKEN_DIM), jnp.float32),
            jax.ShapeDtypeStruct((B, N), jnp.float32),
        ],
        compiler_params=pltpu.CompilerParams(
            allow_input_fusion=[True, False, False],
        ),
    )(x, wt, b2)

    X = out.reshape(B, L, N, TOKEN_DIM)
    return (X, bi)


def kernel(ecg, W, b):
    return _run(ecg, W, b)
